# baseline (speedup 1.0000x reference)
import jax
import jax.numpy as jnp
from jax.experimental import pallas as pl
from jax.experimental.pallas import tpu as pltpu

N = 1024
D = 128
TILE = 128
NT = N // TILE
GRP = 128
EPS = 1e-7
MAXN = 0.99999
CLIP_HI = 1.0 - 1e-6


def _row_tree(acc):
    s04 = acc[0:1] + acc[4:5]
    s26 = acc[2:3] + acc[6:7]
    s15 = acc[1:2] + acc[5:6]
    s37 = acc[3:4] + acc[7:8]
    return (s04 + s26) + (s15 + s37)


def _r2t(terms):
    acc = terms[0:8, :]
    for t in range(1, 16):
        acc = acc + terms[8 * t:8 * t + 8, :]
    return _row_tree(acc)


def _r2t_mul(a, b):
    acc = a[0:8, :] * b[0:8, :]
    for t in range(1, 16):
        acc = acc + a[8 * t:8 * t + 8, :] * b[8 * t:8 * t + 8, :]
    return _row_tree(acc)


def _agg_kernel(x_ref, xt_ref, y2_ref, adj_ref, o_ref):
    i = pl.program_id(0)
    xk8 = x_ref[pl.ds(i * GRP, GRP), :]
    arow8 = adj_ref[pl.ds(i * GRP, GRP), :]
    xcols = jnp.transpose(xk8)

    xks = [xk8[j:j + 1, :] for j in range(GRP)]
    x2s = [jnp.sum(xk * xk) for xk in xks]
    bcoefs = [1.0 - x2 for x2 in x2s]
    coefgs = [jnp.maximum(jnp.float32(EPS), b) for b in bcoefs]
    ncols = -xcols

    accs = [None] * GRP
    for u in range(NT):
        txu = xt_ref[:, u * TILE:(u + 1) * TILE]
        y2 = y2_ref[:, u * TILE:(u + 1) * TILE]
        for j in range(GRP):
            nxkT = jnp.broadcast_to(ncols[:, j:j + 1], (D, TILE))
            xy = _r2t_mul(nxkT, txu)
            add88 = xy * 2.0 + 1.0
            acoef = add88 + y2
            den = jnp.maximum(add88 + x2s[j] * y2, EPS)
            sub = (acoef * nxkT + bcoefs[j] * txu) / den
            sn2 = _r2t_mul(sub, sub)
            subn = jnp.maximum(EPS, jnp.sqrt(sn2))
            arg = jnp.clip(subn, 0.0, CLIP_HI)
            at = (jnp.log1p(arg) - jnp.log1p(-arg)) * 0.5
            g = coefgs[j] * at
            m = (arow8[j:j + 1, u * TILE:(u + 1) * TILE] > 0.0).astype(
                jnp.float32)
            logs_t = ((g * sub) / subn) * m
            logs_n = logs_t.T
            acc = accs[j]
            for v_i in range(16):
                blk = logs_n[8 * v_i:8 * v_i + 8, :]
                acc = blk if acc is None else acc + blk
            accs[j] = acc

    for j in range(GRP):
        xk, x2, bcoef, coefg = xks[j], x2s[j], bcoefs[j], coefgs[j]
        v = _row_tree(accs[j])
        vn2 = jnp.sum(v * v)
        vn = jnp.maximum(jnp.float32(EPS), jnp.sqrt(vn2))
        th = jnp.tanh((2.0 / coefg) * vn * 0.5)
        second = (th * v) / vn
        y2e = jnp.sum(second * second)
        xye = jnp.sum(xk * second)
        add90 = xye * 2.0 + 1.0
        acoefe = add90 + y2e
        dene = jnp.maximum(add90 + x2 * y2e, EPS)
        nume = acoefe * xk + bcoef * second
        res = nume / dene
        rn2 = jnp.sum(res * res)
        nrm = jnp.sqrt(rn2)
        maxnrm = jnp.maximum(nrm, jnp.float32(EPS))
        alt = (nume / (dene * maxnrm)) * MAXN
        o_ref[j:j + 1, :] = jnp.where(nrm > MAXN, alt, res)


def kernel(x, adj):
    xt = x.T
    y2_all = jnp.sum(x * x, axis=1).reshape(1, N)
    return pl.pallas_call(
        _agg_kernel,
        grid=(N // GRP,),
        in_specs=[
            pl.BlockSpec((N, D), lambda i: (0, 0)),
            pl.BlockSpec((D, N), lambda i: (0, 0)),
            pl.BlockSpec((1, N), lambda i: (0, 0)),
            pl.BlockSpec((N, N), lambda i: (0, 0)),
        ],
        out_specs=pl.BlockSpec((GRP, D), lambda i: (i, 0)),
        out_shape=jax.ShapeDtypeStruct((N, D), jnp.float32),
        compiler_params=pltpu.CompilerParams(
            dimension_semantics=("parallel",),
        ),
    )(x, xt, y2_all, adj)


# speedup vs baseline: 1.1885x; 1.1885x over previous
import jax
import jax.numpy as jnp
from jax.experimental import pallas as pl
from jax.experimental.pallas import tpu as pltpu

N = 1024
D = 128
TILE = 128
NT = N // TILE
GRP = 64
EPS = 1e-7
MAXN = 0.99999
CLIP_HI = 1.0 - 1e-6


def _row_tree(acc):
    s04 = acc[0:1] + acc[4:5]
    s26 = acc[2:3] + acc[6:7]
    s15 = acc[1:2] + acc[5:6]
    s37 = acc[3:4] + acc[7:8]
    return (s04 + s26) + (s15 + s37)


def _r2t(terms):
    acc = terms[0:8, :]
    for t in range(1, 16):
        acc = acc + terms[8 * t:8 * t + 8, :]
    return _row_tree(acc)


def _r2t_mul(a, b):
    acc = a[0:8, :] * b[0:8, :]
    for t in range(1, 16):
        acc = acc + a[8 * t:8 * t + 8, :] * b[8 * t:8 * t + 8, :]
    return _row_tree(acc)


def _agg_kernel(x_ref, xt_ref, y2_ref, adj_ref, o_ref):
    i = pl.program_id(0)
    xk8 = x_ref[pl.ds(i * GRP, GRP), :]
    arow8 = adj_ref[pl.ds(i * GRP, GRP), :]
    xcols = jnp.transpose(xk8)

    xks = [xk8[j:j + 1, :] for j in range(GRP)]
    x2s = [jnp.sum(xk * xk) for xk in xks]
    bcoefs = [1.0 - x2 for x2 in x2s]
    coefgs = [jnp.maximum(jnp.float32(EPS), b) for b in bcoefs]
    ncols = -xcols

    accs = [None] * GRP
    for u in range(NT):
        txu = xt_ref[:, u * TILE:(u + 1) * TILE]
        y2 = y2_ref[:, u * TILE:(u + 1) * TILE]
        for j in range(GRP):
            nxkT = jnp.broadcast_to(ncols[:, j:j + 1], (D, TILE))
            xy = _r2t_mul(nxkT, txu)
            add88 = xy * 2.0 + 1.0
            acoef = add88 + y2
            den = jnp.maximum(add88 + x2s[j] * y2, EPS)
            sub = (acoef * nxkT + bcoefs[j] * txu) / den
            sn2 = _r2t_mul(sub, sub)
            subn = jnp.maximum(EPS, jnp.sqrt(sn2))
            arg = jnp.clip(subn, 0.0, CLIP_HI)
            at = (jnp.log1p(arg) - jnp.log1p(-arg)) * 0.5
            g = coefgs[j] * at
            m = (arow8[j:j + 1, u * TILE:(u + 1) * TILE] > 0.0).astype(
                jnp.float32)
            logs_t = ((g * sub) / subn) * m
            logs_n = logs_t.T
            acc = accs[j]
            for v_i in range(16):
                blk = logs_n[8 * v_i:8 * v_i + 8, :]
                acc = blk if acc is None else acc + blk
            accs[j] = acc

    for j in range(GRP):
        xk, x2, bcoef, coefg = xks[j], x2s[j], bcoefs[j], coefgs[j]
        v = _row_tree(accs[j])
        vn2 = jnp.sum(v * v)
        vn = jnp.maximum(jnp.float32(EPS), jnp.sqrt(vn2))
        th = jnp.tanh((2.0 / coefg) * vn * 0.5)
        second = (th * v) / vn
        y2e = jnp.sum(second * second)
        xye = jnp.sum(xk * second)
        add90 = xye * 2.0 + 1.0
        acoefe = add90 + y2e
        dene = jnp.maximum(add90 + x2 * y2e, EPS)
        nume = acoefe * xk + bcoef * second
        res = nume / dene
        rn2 = jnp.sum(res * res)
        nrm = jnp.sqrt(rn2)
        maxnrm = jnp.maximum(nrm, jnp.float32(EPS))
        alt = (nume / (dene * maxnrm)) * MAXN
        o_ref[j:j + 1, :] = jnp.where(nrm > MAXN, alt, res)


def kernel(x, adj):
    xt = x.T
    y2_all = jnp.sum(x * x, axis=1).reshape(1, N)
    return pl.pallas_call(
        _agg_kernel,
        grid=(N // GRP,),
        in_specs=[
            pl.BlockSpec((N, D), lambda i: (0, 0)),
            pl.BlockSpec((D, N), lambda i: (0, 0)),
            pl.BlockSpec((1, N), lambda i: (0, 0)),
            pl.BlockSpec((N, N), lambda i: (0, 0)),
        ],
        out_specs=pl.BlockSpec((GRP, D), lambda i: (i, 0)),
        out_shape=jax.ShapeDtypeStruct((N, D), jnp.float32),
        compiler_params=pltpu.CompilerParams(
            dimension_semantics=("parallel",),
        ),
    )(x, xt, y2_all, adj)
